# baseline (speedup 1.0000x reference)
import jax
import jax.numpy as jnp
from jax.experimental import pallas as pl
from jax.experimental.pallas import tpu as pltpu

_H = 32


def _store_kernel(buf_obs, buf_act, buf_rew, buf_v_next, buf_done,
                  obs, act, rew, v_next, done_mask,
                  out_obs, out_act, out_rew, out_v_next, out_done,
                  sems):
    copies = []
    pairs = (
        (buf_obs, obs, out_obs),
        (buf_act, act, out_act),
        (buf_rew, rew, out_rew),
        (buf_v_next, v_next, out_v_next),
        (buf_done, done_mask, out_done),
    )
    i = 0
    for buf, new, out in pairs:
        copies.append(pltpu.make_async_copy(
            buf.at[pl.ds(1, _H - 1)], out.at[pl.ds(1, _H - 1)], sems.at[i]))
        i += 1
        copies.append(pltpu.make_async_copy(new, out.at[0], sems.at[i]))
        i += 1
    for c in copies:
        c.start()
    for c in copies:
        c.wait()


def kernel(buf_obs, buf_act, buf_rew, buf_v_next, buf_done,
           obs, act, rew, v_next, done_mask):
    h, n = buf_rew.shape
    buf_done_u8 = buf_done.view(jnp.uint8).reshape(h, n // 128, 128)
    done_mask_u8 = done_mask.view(jnp.uint8).reshape(n // 128, 128)
    buf_rew3 = buf_rew.reshape(h, n // 128, 128)
    rew2 = rew.reshape(n // 128, 128)
    buf_v3 = buf_v_next.reshape(h, n // 128, 128)
    v2 = v_next.reshape(n // 128, 128)
    any_spec = pl.BlockSpec(memory_space=pl.ANY)
    out_shapes = (
        jax.ShapeDtypeStruct(buf_obs.shape, buf_obs.dtype),
        jax.ShapeDtypeStruct(buf_act.shape, buf_act.dtype),
        jax.ShapeDtypeStruct(buf_rew3.shape, buf_rew3.dtype),
        jax.ShapeDtypeStruct(buf_v3.shape, buf_v3.dtype),
        jax.ShapeDtypeStruct(buf_done_u8.shape, jnp.uint8),
    )
    out = pl.pallas_call(
        _store_kernel,
        in_specs=[any_spec] * 10,
        out_specs=[any_spec] * 5,
        out_shape=out_shapes,
        scratch_shapes=[pltpu.SemaphoreType.DMA((10,))],
    )(buf_obs, buf_act, buf_rew3, buf_v3, buf_done_u8,
      obs, act, rew2, v2, done_mask_u8)
    return (out[0], out[1], out[2].reshape(h, n), out[3].reshape(h, n),
            out[4].reshape(h, n).view(jnp.bool_))


# speedup vs baseline: 11.5881x; 11.5881x over previous
import jax
import jax.numpy as jnp
from jax.experimental import pallas as pl
from jax.experimental.pallas import tpu as pltpu

_H = 32


def _store_kernel(buf_obs, buf_act, buf_rew, buf_v_next, buf_done,
                  obs, act, rew, v_next, done_mask,
                  out_obs, out_act, out_rew, out_v_next, out_done):
    i = pl.program_id(0)

    @pl.when(i == 0)
    def _():
        out_obs[0] = obs[...]
        out_act[0] = act[...]
        out_rew[...] = buf_rew[...]
        out_rew[0] = rew[...]
        out_v_next[...] = buf_v_next[...]
        out_v_next[0] = v_next[...]
        out_done[...] = buf_done[...]
        out_done[0] = done_mask[...]

    @pl.when(i > 0)
    def _():
        out_obs[0] = buf_obs[0]
        out_act[0] = buf_act[0]


def kernel(buf_obs, buf_act, buf_rew, buf_v_next, buf_done,
           obs, act, rew, v_next, done_mask):
    h = buf_obs.shape[0]
    n_obs = buf_obs.shape[1] * buf_obs.shape[2]
    n_act = buf_act.shape[1] * buf_act.shape[2]
    n = buf_rew.shape[1]

    bo = buf_obs.reshape(h, n_obs // 128, 128)
    ba = buf_act.reshape(h, n_act // 128, 128)
    br = buf_rew.reshape(h, n // 128, 128)
    bv = buf_v_next.reshape(h, n // 128, 128)
    bd = buf_done.view(jnp.uint8).reshape(h, n // 128, 128)
    o2 = obs.reshape(n_obs // 128, 128)
    a2 = act.reshape(n_act // 128, 128)
    r2 = rew.reshape(n // 128, 128)
    v2 = v_next.reshape(n // 128, 128)
    d2 = done_mask.view(jnp.uint8).reshape(n // 128, 128)

    row = lambda arr: pl.BlockSpec(
        (1,) + arr.shape[1:], lambda i: (i, 0, 0))
    row_clamped = lambda arr: pl.BlockSpec(
        (1,) + arr.shape[1:], lambda i: (jnp.maximum(i, 1), 0, 0))
    whole3 = lambda arr: pl.BlockSpec(arr.shape, lambda i: (0, 0, 0))
    whole2 = lambda arr: pl.BlockSpec(arr.shape, lambda i: (0, 0))

    out_shapes = (
        jax.ShapeDtypeStruct(bo.shape, bo.dtype),
        jax.ShapeDtypeStruct(ba.shape, ba.dtype),
        jax.ShapeDtypeStruct(br.shape, br.dtype),
        jax.ShapeDtypeStruct(bv.shape, bv.dtype),
        jax.ShapeDtypeStruct(bd.shape, bd.dtype),
    )
    out = pl.pallas_call(
        _store_kernel,
        grid=(h,),
        in_specs=[row_clamped(bo), row_clamped(ba),
                  whole3(br), whole3(bv), whole3(bd),
                  whole2(o2), whole2(a2), whole2(r2), whole2(v2), whole2(d2)],
        out_specs=[row(bo), row(ba), whole3(br), whole3(bv), whole3(bd)],
        out_shape=out_shapes,
    )(bo, ba, br, bv, bd, o2, a2, r2, v2, d2)
    return (out[0].reshape(buf_obs.shape), out[1].reshape(buf_act.shape),
            out[2].reshape(h, n), out[3].reshape(h, n),
            out[4].reshape(h, n).view(jnp.bool_))


# speedup vs baseline: 27.4833x; 2.3717x over previous
import jax
import jax.numpy as jnp
from jax.experimental import pallas as pl
from jax.experimental.pallas import tpu as pltpu


def _store_kernel(buf_obs, buf_act, buf_rew, buf_v_next, buf_done,
                  obs, act, rew, v_next, done_mask,
                  out_obs, out_act, out_rew, out_v_next, out_done):
    i = pl.program_id(0)

    @pl.when(i == 0)
    def _():
        out_obs[0] = obs[...]
        out_act[0] = act[...]
        out_rew[...] = buf_rew[...]
        out_rew[0] = rew[...]
        out_v_next[...] = buf_v_next[...]
        out_v_next[0] = v_next[...]
        out_done[...] = buf_done[...]
        out_done[0] = done_mask[...]

    @pl.when(i > 0)
    def _():
        out_obs[0] = buf_obs[0]
        out_act[0] = buf_act[0]


def kernel(buf_obs, buf_act, buf_rew, buf_v_next, buf_done,
           obs, act, rew, v_next, done_mask):
    h = buf_obs.shape[0]
    bd = buf_done.view(jnp.uint8)
    d1 = done_mask.view(jnp.uint8)

    row = lambda arr: pl.BlockSpec(
        (1,) + arr.shape[1:], lambda i: (i, 0, 0))
    row_clamped = lambda arr: pl.BlockSpec(
        (1,) + arr.shape[1:], lambda i: (jnp.maximum(i, 1), 0, 0))
    whole = lambda arr: pl.BlockSpec(
        arr.shape, lambda i: (0,) * arr.ndim)

    out_shapes = (
        jax.ShapeDtypeStruct(buf_obs.shape, buf_obs.dtype),
        jax.ShapeDtypeStruct(buf_act.shape, buf_act.dtype),
        jax.ShapeDtypeStruct(buf_rew.shape, buf_rew.dtype),
        jax.ShapeDtypeStruct(buf_v_next.shape, buf_v_next.dtype),
        jax.ShapeDtypeStruct(bd.shape, bd.dtype),
    )
    out = pl.pallas_call(
        _store_kernel,
        grid=(h,),
        in_specs=[row_clamped(buf_obs), row_clamped(buf_act),
                  whole(buf_rew), whole(buf_v_next), whole(bd),
                  whole(obs), whole(act), whole(rew), whole(v_next),
                  whole(d1)],
        out_specs=[row(buf_obs), row(buf_act),
                   whole(buf_rew), whole(buf_v_next), whole(bd)],
        out_shape=out_shapes,
    )(buf_obs, buf_act, buf_rew, buf_v_next, bd,
      obs, act, rew, v_next, d1)
    return (out[0], out[1], out[2], out[3], out[4].view(jnp.bool_))


# speedup vs baseline: 27.7772x; 1.0107x over previous
import jax
import jax.numpy as jnp
from jax.experimental import pallas as pl
from jax.experimental.pallas import tpu as pltpu

_NBUF = 6
_K = 4


def _store_kernel(buf_obs, buf_act, buf_rew, buf_v_next, buf_done,
                  obs, act, rew, v_next, done_mask,
                  out_obs, out_act, out_rew, out_v_next, out_done,
                  obs_scr_a, obs_scr_b, act_scr, rew_scr, v_scr, d_scr,
                  rew_new, v_new, d_new,
                  oa_in_sems, oa_out_sems, ob_in_sems, ob_out_sems,
                  act_in_sems, act_out_sems, small_sems):
    h = out_obs.shape[0]
    c = out_obs.shape[2] // 2

    def col_copies(src_new, src_buf, out, scr, in_sems, out_sems, lo):
        cols = pl.ds(lo, c)
        ins, outs = [], []
        for i in range(h):
            src = src_new.at[:, cols] if i == 0 else src_buf.at[i, :, cols]
            ins.append(pltpu.make_async_copy(
                src, scr.at[i % _NBUF], in_sems.at[i % _NBUF]))
            outs.append(pltpu.make_async_copy(
                scr.at[i % _NBUF], out.at[i, :, cols],
                out_sems.at[i % _NBUF]))
        return ins, outs

    oa_in, oa_out = col_copies(obs, buf_obs, out_obs, obs_scr_a,
                               oa_in_sems, oa_out_sems, 0)
    ob_in, ob_out = col_copies(obs, buf_obs, out_obs, obs_scr_b,
                               ob_in_sems, ob_out_sems, c)
    act_in = [pltpu.make_async_copy(
        act if i == 0 else buf_act.at[i],
        act_scr.at[i % _NBUF], act_in_sems.at[i % _NBUF]) for i in range(h)]
    act_out = [pltpu.make_async_copy(
        act_scr.at[i % _NBUF], out_act.at[i], act_out_sems.at[i % _NBUF])
        for i in range(h)]

    ins = [oa_in, ob_in, act_in]
    outs = [oa_out, ob_out, act_out]

    sm_in = [pltpu.make_async_copy(src, scr, small_sems.at[j])
             for j, (src, scr) in
             enumerate(((buf_rew, rew_scr), (buf_v_next, v_scr),
                        (buf_done, d_scr), (rew, rew_new),
                        (v_next, v_new), (done_mask, d_new)))]
    sm_out = [pltpu.make_async_copy(scr, dst, small_sems.at[6 + j])
              for j, (scr, dst) in
              enumerate(((rew_scr, out_rew), (v_scr, out_v_next),
                         (d_scr, out_done)))]

    for cop in sm_in:
        cop.start()
    for i in range(_K):
        for s in ins:
            s[i].start()

    for i in range(h):
        j = i + _K
        if j < h:
            if j - _NBUF >= 0:
                for s in outs:
                    s[j - _NBUF].wait()
            for s in ins:
                s[j].start()
        for s in ins:
            s[i].wait()
        for s in outs:
            s[i].start()
        if i == 1:
            for cop in sm_in:
                cop.wait()
            rew_scr[0] = rew_new[...]
            v_scr[0] = v_new[...]
            d_scr[0] = d_new[...]
            for cop in sm_out:
                cop.start()

    for i in range(h - _NBUF, h):
        for s in outs:
            s[i].wait()
    for cop in sm_out:
        cop.wait()


def kernel(buf_obs, buf_act, buf_rew, buf_v_next, buf_done,
           obs, act, rew, v_next, done_mask):
    bd = buf_done.view(jnp.uint8)
    d1 = done_mask.view(jnp.uint8)
    c = buf_obs.shape[2] // 2

    any_spec = pl.BlockSpec(memory_space=pl.ANY)
    out_shapes = (
        jax.ShapeDtypeStruct(buf_obs.shape, buf_obs.dtype),
        jax.ShapeDtypeStruct(buf_act.shape, buf_act.dtype),
        jax.ShapeDtypeStruct(buf_rew.shape, buf_rew.dtype),
        jax.ShapeDtypeStruct(buf_v_next.shape, buf_v_next.dtype),
        jax.ShapeDtypeStruct(bd.shape, bd.dtype),
    )
    out = pl.pallas_call(
        _store_kernel,
        in_specs=[any_spec] * 10,
        out_specs=[any_spec] * 5,
        out_shape=out_shapes,
        scratch_shapes=[
            pltpu.VMEM((_NBUF, buf_obs.shape[1], c), buf_obs.dtype),
            pltpu.VMEM((_NBUF, buf_obs.shape[1], c), buf_obs.dtype),
            pltpu.VMEM((_NBUF,) + buf_act.shape[1:], buf_act.dtype),
            pltpu.VMEM(buf_rew.shape, buf_rew.dtype),
            pltpu.VMEM(buf_v_next.shape, buf_v_next.dtype),
            pltpu.VMEM(bd.shape, bd.dtype),
            pltpu.VMEM(rew.shape, rew.dtype),
            pltpu.VMEM(v_next.shape, v_next.dtype),
            pltpu.VMEM(d1.shape, d1.dtype),
            pltpu.SemaphoreType.DMA((_NBUF,)),
            pltpu.SemaphoreType.DMA((_NBUF,)),
            pltpu.SemaphoreType.DMA((_NBUF,)),
            pltpu.SemaphoreType.DMA((_NBUF,)),
            pltpu.SemaphoreType.DMA((_NBUF,)),
            pltpu.SemaphoreType.DMA((_NBUF,)),
            pltpu.SemaphoreType.DMA((9,)),
        ],
    )(buf_obs, buf_act, buf_rew, buf_v_next, bd,
      obs, act, rew, v_next, d1)
    return (out[0], out[1], out[2], out[3], out[4].view(jnp.bool_))


# speedup vs baseline: 27.8612x; 1.0030x over previous
import jax
import jax.numpy as jnp
from jax.experimental import pallas as pl
from jax.experimental.pallas import tpu as pltpu

_NBUF = 6
_K = 4


def _store_kernel(buf_obs, buf_act, buf_rew, buf_v_next, buf_done,
                  obs, act, rew, v_next, done_mask,
                  out_obs, out_act, out_rew, out_v_next, out_done,
                  obs_scr, act_scr, rew_scr, v_scr, d_scr,
                  rew_new, v_new, d_new,
                  obs_in_sems, obs_out_sems, act_in_sems, act_out_sems,
                  small_sems):
    h = out_obs.shape[0]

    def obs_src(i):
        return obs if i == 0 else buf_obs.at[i]

    def act_src(i):
        return act if i == 0 else buf_act.at[i]

    obs_in = [pltpu.make_async_copy(
        obs_src(i), obs_scr.at[i % _NBUF], obs_in_sems.at[i % _NBUF])
        for i in range(h)]
    obs_out = [pltpu.make_async_copy(
        obs_scr.at[i % _NBUF], out_obs.at[i], obs_out_sems.at[i % _NBUF])
        for i in range(h)]
    act_in = [pltpu.make_async_copy(
        act_src(i), act_scr.at[i % _NBUF], act_in_sems.at[i % _NBUF])
        for i in range(h)]
    act_out = [pltpu.make_async_copy(
        act_scr.at[i % _NBUF], out_act.at[i], act_out_sems.at[i % _NBUF])
        for i in range(h)]

    sm_in = [pltpu.make_async_copy(src, scr, small_sems.at[j]) for j, (src, scr) in
             enumerate(((buf_rew, rew_scr), (buf_v_next, v_scr),
                        (buf_done, d_scr)))]
    sm_patch = [pltpu.make_async_copy(src, scr, small_sems.at[3 + j])
                for j, (src, scr) in
                enumerate(((rew, rew_new), (v_next, v_new),
                           (done_mask, d_new)))]
    sm_out = [pltpu.make_async_copy(scr, dst, small_sems.at[6 + j])
              for j, (scr, dst) in
              enumerate(((rew_scr, out_rew), (v_scr, out_v_next),
                         (d_scr, out_done)))]

    for c in sm_in:
        c.start()
    for i in range(_K):
        obs_in[i].start()
        act_in[i].start()

    for i in range(h):
        j = i + _K
        if j < h:
            if j - _NBUF >= 0:
                obs_out[j - _NBUF].wait()
                act_out[j - _NBUF].wait()
            obs_in[j].start()
            act_in[j].start()
        obs_in[i].wait()
        act_in[i].wait()
        obs_out[i].start()
        act_out[i].start()
        if i == 1:
            for c in sm_in:
                c.wait()
            for c in sm_patch:
                c.start()
        if i == 3:
            for c in sm_patch:
                c.wait()
            rew_scr[0] = rew_new[...]
            v_scr[0] = v_new[...]
            d_scr[0] = d_new[...]
            for c in sm_out:
                c.start()

    for i in range(h - _NBUF, h):
        obs_out[i].wait()
        act_out[i].wait()
    for c in sm_out:
        c.wait()


def kernel(buf_obs, buf_act, buf_rew, buf_v_next, buf_done,
           obs, act, rew, v_next, done_mask):
    h = buf_obs.shape[0]
    bd = buf_done.view(jnp.uint8)
    d1 = done_mask.view(jnp.uint8)

    any_spec = pl.BlockSpec(memory_space=pl.ANY)
    out_shapes = (
        jax.ShapeDtypeStruct(buf_obs.shape, buf_obs.dtype),
        jax.ShapeDtypeStruct(buf_act.shape, buf_act.dtype),
        jax.ShapeDtypeStruct(buf_rew.shape, buf_rew.dtype),
        jax.ShapeDtypeStruct(buf_v_next.shape, buf_v_next.dtype),
        jax.ShapeDtypeStruct(bd.shape, bd.dtype),
    )
    out = pl.pallas_call(
        _store_kernel,
        in_specs=[any_spec] * 10,
        out_specs=[any_spec] * 5,
        out_shape=out_shapes,
        scratch_shapes=[
            pltpu.VMEM((_NBUF,) + buf_obs.shape[1:], buf_obs.dtype),
            pltpu.VMEM((_NBUF,) + buf_act.shape[1:], buf_act.dtype),
            pltpu.VMEM(buf_rew.shape, buf_rew.dtype),
            pltpu.VMEM(buf_v_next.shape, buf_v_next.dtype),
            pltpu.VMEM(bd.shape, bd.dtype),
            pltpu.VMEM(rew.shape, rew.dtype),
            pltpu.VMEM(v_next.shape, v_next.dtype),
            pltpu.VMEM(d1.shape, d1.dtype),
            pltpu.SemaphoreType.DMA((_NBUF,)),
            pltpu.SemaphoreType.DMA((_NBUF,)),
            pltpu.SemaphoreType.DMA((_NBUF,)),
            pltpu.SemaphoreType.DMA((_NBUF,)),
            pltpu.SemaphoreType.DMA((9,)),
        ],
    )(buf_obs, buf_act, buf_rew, buf_v_next, bd,
      obs, act, rew, v_next, d1)
    return (out[0], out[1], out[2], out[3], out[4].view(jnp.bool_))
